# Initial kernel scaffold; baseline (speedup 1.0000x reference)
import jax
import jax.numpy as jnp
from jax.experimental import pallas as pl


def kernel(magno_patches, vit_positional_embedding, scores):
    raise NotImplementedError("write your pallas kernel here")


# speedup vs baseline: 8.3449x; 8.3449x over previous
import functools

import jax
import jax.numpy as jnp
from jax import lax
from jax.experimental import pallas as pl
from jax.experimental.pallas import tpu as pltpu
from jax.experimental.pallas import tpu_sc as plsc


def _topk_body(k, s_ref, st_ref, o_ref):
    row = s_ref[0]
    col = st_ref[0]
    n = row.shape[1]
    ii = lax.broadcasted_iota(jnp.int32, (n, n), 0)
    jj = lax.broadcasted_iota(jnp.int32, (n, n), 1)
    gt = (row > col).astype(jnp.int32)
    eq_lt = ((row == col) & (jj < ii)).astype(jnp.int32)
    rank = jnp.sum(gt + eq_lt, axis=1, keepdims=True)
    rr = lax.broadcasted_iota(jnp.int32, (n, k), 1)
    ivals = lax.broadcasted_iota(jnp.int32, (n, k), 0)
    onehot = rank == rr
    o_ref[0] = jnp.sum(jnp.where(onehot, ivals, 0), axis=0, keepdims=True)


def _topk_indices(scores, k):
    b, n = scores.shape
    s3 = scores.reshape(b, 1, n)
    st3 = jnp.swapaxes(s3, 1, 2)
    return pl.pallas_call(
        functools.partial(_topk_body, k),
        grid=(b,),
        in_specs=[
            pl.BlockSpec((1, 1, n), lambda i: (i, 0, 0)),
            pl.BlockSpec((1, n, 1), lambda i: (i, 0, 0)),
        ],
        out_specs=pl.BlockSpec((1, 1, k), lambda i: (i, 0, 0)),
        out_shape=jax.ShapeDtypeStruct((b, 1, k), jnp.int32),
    )(s3, st3).reshape(b, k)


_CHUNK = 64


def _sc_gather_add(idx, patches_flat, pos_table, k, d):
    b, _ = idx.shape
    n_chunks = k // _CHUNK
    mesh = plsc.VectorSubcoreMesh(core_axis_name="c", subcore_axis_name="s")

    @functools.partial(
        pl.kernel,
        mesh=mesh,
        out_type=jax.ShapeDtypeStruct((b * k, d), jnp.float32),
        scratch_types=[
            pltpu.VMEM((k,), jnp.int32),
            pltpu.VMEM((n_chunks, _CHUNK), jnp.int32),
            pltpu.VMEM((n_chunks, _CHUNK), jnp.int32),
            pltpu.VMEM((_CHUNK, d), jnp.float32),
            pltpu.VMEM((_CHUNK, d), jnp.float32),
            pltpu.SemaphoreType.DMA,
            pltpu.SemaphoreType.DMA,
        ],
    )
    def sc_kernel(idx_hbm, patches_hbm, pos_hbm, out_hbm,
                  idxrow_v, fidx_v, pidx_v, pbuf, qbuf, sem1, sem2):
        wid = lax.axis_index("s") * 2 + lax.axis_index("c")
        pltpu.sync_copy(idx_hbm.at[wid], idxrow_v)
        base = wid * 1024
        for c in range(k // 16):
            v = idxrow_v[pl.ds(c * 16, 16)]
            g = c // (_CHUNK // 16)
            r = (c % (_CHUNK // 16)) * 16
            fidx_v[g, pl.ds(r, 16)] = v + base
            pidx_v[g, pl.ds(r, 16)] = v + 1
        for g in range(n_chunks):
            cp1 = pltpu.make_async_copy(patches_hbm.at[fidx_v.at[g]], pbuf, sem1)
            cp2 = pltpu.make_async_copy(pos_hbm.at[pidx_v.at[g]], qbuf, sem2)
            cp1.start()
            cp2.start()
            cp1.wait()
            cp2.wait()

            def body(r, carry):
                for c in range(d // 16):
                    sl = pl.ds(c * 16, 16)
                    pbuf[r, sl] = pbuf[r, sl] + qbuf[r, sl]
                return carry

            lax.fori_loop(0, _CHUNK, body, 0)
            pltpu.sync_copy(pbuf, out_hbm.at[pl.ds(wid * k + g * _CHUNK, _CHUNK)])

    return sc_kernel(idx, patches_flat, pos_table)


def kernel(magno_patches, vit_positional_embedding, scores):
    b, n, d = magno_patches.shape
    k = n // 4
    idx = _topk_indices(scores, k)
    patches_flat = magno_patches.reshape(b * n, d)
    pos_table = vit_positional_embedding.reshape(n + 1, d)
    out = _sc_gather_add(idx, patches_flat, pos_table, k, d)
    return out.reshape(b, k, d)
